# baseline (speedup 1.0000x reference)
import functools

import jax
import jax.numpy as jnp
from jax import lax
from jax.experimental import pallas as pl
from jax.experimental.pallas import tpu as pltpu
from jax.experimental.pallas import tpu_sc as plsc

HIDDEN = 768
MAX_SEQ = 2048
B_TOTAL = 8192
NW = 32
B_PER_W = B_TOTAL // NW
CHUNK = 32
NCHUNK = B_PER_W // CHUNK
LANES = 16
GROUPS = HIDDEN // 32
PACKED = HIDDEN // 2


def _pack_body(pos_ref, out_ref):
    x = lax.bitcast_convert_type(pos_ref[...], jnp.int32)
    r = lax.shift_right_logical(
        x + 0x7FFF + (lax.shift_right_logical(x, 16) & 1), 16)
    v = r.reshape(pos_ref.shape[0], GROUPS, 32)
    w = v[:, :, :16] | lax.shift_left(v[:, :, 16:], 16)
    out_ref[...] = w.reshape(pos_ref.shape[0], PACKED)


def _emb_body(widx_hbm, pidx_hbm, word_hbm, posp_hbm, out_hbm,
              idx_w, idx_p, bw, bp,
              sem_w0, sem_w1, sem_p0, sem_p1, sem_o0, sem_o1):
    wid = lax.axis_index("s") * 2 + lax.axis_index("c")
    base = wid * B_PER_W
    pltpu.sync_copy(widx_hbm.at[pl.ds(base, B_PER_W)], idx_w)
    pltpu.sync_copy(pidx_hbm.at[pl.ds(base, B_PER_W)], idx_p)

    sems_w = (sem_w0, sem_w1)
    sems_p = (sem_p0, sem_p1)
    sems_o = (sem_o0, sem_o1)
    gath = [None, None]
    outd = [None, None]

    for c in range(NCHUNK + 1):
        k = c % 2
        if c < NCHUNK:
            if outd[k] is not None:
                outd[k].wait()
            gath[k] = (
                pltpu.async_copy(
                    word_hbm.at[idx_w.at[pl.ds(c * CHUNK, CHUNK)]],
                    bw.at[k], sems_w[k]),
                pltpu.async_copy(
                    posp_hbm.at[idx_p.at[pl.ds(c * CHUNK, CHUNK)]],
                    bp.at[k], sems_p[k]),
            )
        if c >= 1:
            kp = (c - 1) % 2
            gath[kp][0].wait()
            gath[kp][1].wait()

            sh16 = jnp.full((LANES,), 16, jnp.int32)
            mhi = jnp.full((LANES,), -65536, jnp.int32)

            @plsc.parallel_loop(0, CHUNK, step=1)
            def row_body(r, kp=kp, sh16=sh16, mhi=mhi):
                for g in range(GROUPS):
                    x = bp.at[kp][r, pl.ds(g * LANES, LANES)]
                    lo = lax.bitcast_convert_type(
                        lax.shift_left(x, sh16), jnp.float32)
                    hi = lax.bitcast_convert_type(
                        lax.bitwise_and(x, mhi), jnp.float32)
                    plsc.addupdate(
                        bw.at[kp].at[r, pl.ds(g * 32, LANES)], lo)
                    plsc.addupdate(
                        bw.at[kp].at[r, pl.ds(g * 32 + LANES, LANES)], hi)

            outd[kp] = pltpu.async_copy(
                bw.at[kp],
                out_hbm.at[pl.ds(base + (c - 1) * CHUNK, CHUNK)],
                sems_o[kp])
    for k in range(2):
        if outd[k] is not None:
            outd[k].wait()


@jax.jit
def _run(widx, pidx, word_table, pos_table):
    pos_pack = pl.pallas_call(
        _pack_body,
        out_shape=jax.ShapeDtypeStruct((MAX_SEQ, PACKED), jnp.int32),
    )(pos_table)
    mesh = plsc.VectorSubcoreMesh(core_axis_name="c", subcore_axis_name="s")
    k = functools.partial(
        pl.kernel,
        mesh=mesh,
        out_type=jax.ShapeDtypeStruct((B_TOTAL, HIDDEN), jnp.float32),
        scratch_types=[
            pltpu.VMEM((B_PER_W,), jnp.int32),
            pltpu.VMEM((B_PER_W,), jnp.int32),
            pltpu.VMEM((2, CHUNK, HIDDEN), jnp.float32),
            pltpu.VMEM((2, CHUNK, PACKED), jnp.int32),
            pltpu.SemaphoreType.DMA,
            pltpu.SemaphoreType.DMA,
            pltpu.SemaphoreType.DMA,
            pltpu.SemaphoreType.DMA,
            pltpu.SemaphoreType.DMA,
            pltpu.SemaphoreType.DMA,
        ],
    )(_emb_body)
    return k(widx, pidx, word_table, pos_pack)


def kernel(input_indices, position_indices, word_table, pos_table):
    widx = input_indices.reshape(-1).astype(jnp.int32)
    pidx = position_indices.reshape(-1).astype(jnp.int32)
    out = _run(widx, pidx, word_table, pos_table)
    return out.reshape(input_indices.shape + (HIDDEN,))


# speedup vs baseline: 1.2878x; 1.2878x over previous
import functools

import jax
import jax.numpy as jnp
from jax import lax
from jax.experimental import pallas as pl
from jax.experimental.pallas import tpu as pltpu
from jax.experimental.pallas import tpu_sc as plsc

HIDDEN = 768
B_TOTAL = 8192
NW = 32
B_PER_W = B_TOTAL // NW
CHUNK = 32
NCHUNK = B_PER_W // CHUNK
LANES = 16
COLS = HIDDEN // LANES


def _emb_body(widx_hbm, pidx_hbm, word_hbm, pos_hbm, out_hbm,
              idx_w, idx_p, bw, bp,
              sem_w0, sem_w1, sem_p0, sem_p1, sem_o0, sem_o1):
    wid = lax.axis_index("s") * 2 + lax.axis_index("c")
    base = wid * B_PER_W
    pltpu.sync_copy(widx_hbm.at[pl.ds(base, B_PER_W)], idx_w)
    pltpu.sync_copy(pidx_hbm.at[pl.ds(base, B_PER_W)], idx_p)

    sems_w = (sem_w0, sem_w1)
    sems_p = (sem_p0, sem_p1)
    sems_o = (sem_o0, sem_o1)
    gath = [None, None]
    outd = [None, None]

    for c in range(NCHUNK + 1):
        k = c % 2
        if c < NCHUNK:
            if outd[k] is not None:
                outd[k].wait()
            gath[k] = (
                pltpu.async_copy(
                    word_hbm.at[idx_w.at[pl.ds(c * CHUNK, CHUNK)]],
                    bw.at[k], sems_w[k]),
                pltpu.async_copy(
                    pos_hbm.at[idx_p.at[pl.ds(c * CHUNK, CHUNK)]],
                    bp.at[k], sems_p[k]),
            )
        if c >= 1:
            kp = (c - 1) % 2
            gath[kp][0].wait()
            gath[kp][1].wait()

            @plsc.parallel_loop(0, CHUNK, step=1)
            def row_body(r, kp=kp):
                for j in range(COLS):
                    sl = (r, pl.ds(j * LANES, LANES))
                    plsc.addupdate(bw.at[kp].at[sl], bp.at[kp][sl])

            outd[kp] = pltpu.async_copy(
                bw.at[kp],
                out_hbm.at[pl.ds(base + (c - 1) * CHUNK, CHUNK)],
                sems_o[kp])
    for k in range(2):
        if outd[k] is not None:
            outd[k].wait()


@jax.jit
def _run(widx, pidx, word_table, pos_table):
    mesh = plsc.VectorSubcoreMesh(core_axis_name="c", subcore_axis_name="s")
    k = functools.partial(
        pl.kernel,
        mesh=mesh,
        out_type=jax.ShapeDtypeStruct((B_TOTAL, HIDDEN), jnp.float32),
        scratch_types=[
            pltpu.VMEM((B_PER_W,), jnp.int32),
            pltpu.VMEM((B_PER_W,), jnp.int32),
            pltpu.VMEM((2, CHUNK, HIDDEN), jnp.float32),
            pltpu.VMEM((2, CHUNK, HIDDEN), jnp.float32),
            pltpu.SemaphoreType.DMA,
            pltpu.SemaphoreType.DMA,
            pltpu.SemaphoreType.DMA,
            pltpu.SemaphoreType.DMA,
            pltpu.SemaphoreType.DMA,
            pltpu.SemaphoreType.DMA,
        ],
    )(_emb_body)
    return k(widx, pidx, word_table, pos_table)


def kernel(input_indices, position_indices, word_table, pos_table):
    widx = input_indices.reshape(-1).astype(jnp.int32)
    pidx = position_indices.reshape(-1).astype(jnp.int32)
    out = _run(widx, pidx, word_table, pos_table)
    return out.reshape(input_indices.shape + (HIDDEN,))
